# baseline (speedup 1.0000x reference)
import functools

import jax
import jax.numpy as jnp
from jax import lax
from jax.experimental import pallas as pl
from jax.experimental.pallas import tpu as pltpu
from jax.experimental.pallas import tpu_sc as plsc

D_TOKEN = 64
BATCH = 4096
HIST = 200
VOCAB = 100

NUM_CORES = 2
NUM_SUBCORES = 16
NW = NUM_CORES * NUM_SUBCORES
BCOL = BATCH // NW
LANES = 16
NBG = BCOL // LANES
TROW = D_TOKEN + 1
NSLOT = 4
N_QUADS = HIST // NSLOT
N_HGROUPS = HIST // 8


@functools.partial(
    pl.kernel,
    out_type=jax.ShapeDtypeStruct((HIST, D_TOKEN, BATCH), jnp.float32),
    mesh=plsc.VectorSubcoreMesh(core_axis_name="c", subcore_axis_name="s"),
    compiler_params=pltpu.CompilerParams(
        use_tc_tiling_on_sc=True, needs_layout_passes=False
    ),
    scratch_types=[
        pltpu.VMEM((VOCAB * TROW,), jnp.float32),
        pltpu.VMEM((2, 8, BCOL), jnp.int32),
        pltpu.VMEM((NSLOT, D_TOKEN, BCOL), jnp.float32),
        pltpu.SemaphoreType.DMA((2,)),
        pltpu.SemaphoreType.DMA((NSLOT,)),
    ],
)
def _sc_emb(table_hbm, xt_hbm, out_hbm, tab_v, idx_v, p_v, isem, osem):
    wid = lax.axis_index("s") * NUM_CORES + lax.axis_index("c")
    col = wid * BCOL
    pltpu.sync_copy(table_hbm, tab_v)

    def fetch_idx(g, s):
        pltpu.async_copy(
            xt_hbm.at[pl.ds(g * 8, 8), pl.ds(col, BCOL)], idx_v.at[s], isem.at[s]
        )

    def wait_idx(g, s):
        pltpu.make_async_copy(
            xt_hbm.at[pl.ds(g * 8, 8), pl.ds(col, BCOL)], idx_v.at[s], isem.at[s]
        ).wait()

    fetch_idx(0, 0)
    fetch_idx(1, 1)

    def compute_block(gslot, r, slot):
        for bg in range(NBG):
            iv = idx_v[gslot, r, pl.ds(bg * LANES, LANES)]
            base = iv * TROW

            @plsc.parallel_loop(0, D_TOKEN, unroll=8)
            def _(d):
                p_v[slot, d, pl.ds(bg * LANES, LANES)] = plsc.load_gather(
                    tab_v, [base + d]
                )

    def quad_body(q, _):
        g = q // 2

        @pl.when(q % 2 == 0)
        def _():
            wait_idx(g, g % 2)

            @pl.when((g >= 1) & (g + 1 < N_HGROUPS))
            def _():
                fetch_idx(g + 1, (g + 1) % 2)

        for k in range(NSLOT):
            h = q * NSLOT + k
            r = (q % 2) * NSLOT + k

            @pl.when(q > 0)
            def _():
                pltpu.make_async_copy(
                    p_v.at[k], out_hbm.at[h].at[:, pl.ds(col, BCOL)], osem.at[k]
                ).wait()

            compute_block(g % 2, r, k)
            pltpu.async_copy(
                p_v.at[k], out_hbm.at[h].at[:, pl.ds(col, BCOL)], osem.at[k]
            )

        return ()

    lax.fori_loop(0, N_QUADS, quad_body, ())

    for k in range(NSLOT):
        pltpu.make_async_copy(
            p_v.at[k],
            out_hbm.at[HIST - NSLOT + k].at[:, pl.ds(col, BCOL)],
            osem.at[k],
        ).wait()


def kernel(x, dic):
    table = jnp.concatenate([dic[-1:], dic[:-1]], axis=0)
    table = jnp.pad(table, ((0, 0), (0, TROW - D_TOKEN))).reshape(VOCAB * TROW)
    p = _sc_emb(table, x.T)
    return p.transpose(2, 0, 1)
